# baseline (speedup 1.0000x reference)
import functools

import jax
import jax.numpy as jnp
from jax import lax
from jax.experimental import pallas as pl
from jax.experimental.pallas import tpu as pltpu
from jax.experimental.pallas import tpu_sc as plsc

B = 16
MAX_LEN = 2048
D = 1024

NC = 2
NS = 16
HALF_LEN = MAX_LEN // NC
CHUNK = 64
NCHUNK = HALF_LEN // CHUNK
HB = 32
SPROWS = 896
PHASE_CHUNKS = (range(0, SPROWS // CHUNK), range(SPROWS // CHUNK, NCHUNK))

_mesh = plsc.VectorSubcoreMesh(core_axis_name="c", subcore_axis_name="s")


@functools.partial(
    pl.kernel,
    mesh=_mesh,
    out_type=jax.ShapeDtypeStruct((B * MAX_LEN, D), jnp.float32),
    scratch_types=[
        pltpu.VMEM((16,), jnp.int32),
        pltpu.VMEM((HB,), jnp.int32),
        pltpu.VMEM((HB,), jnp.int32),
        pltpu.VMEM((HB, D), jnp.float32),
        pltpu.VMEM((HB, D), jnp.float32),
        pltpu.VMEM_SHARED((SPROWS, D), jnp.float32),
        pltpu.VMEM_SHARED((CHUNK, D), jnp.float32),
        pltpu.SemaphoreType.DMA,
        pltpu.SemaphoreType.DMA,
    ],
)
def _pos_enc_sc(len_hbm, table_hbm, out_hbm, len_v, idx_a, idx_b, buf_a,
                buf_b, sp_data, sp_pad, gsem, csem):
    c = lax.axis_index("c")
    s = lax.axis_index("s")
    i0 = c * HALF_LEN
    row_out0 = s * MAX_LEN + i0

    lane = lax.iota(jnp.int32, 16)
    idxs = (idx_a, idx_b)
    bufs = (buf_a, buf_b)

    wid = s * NC + c
    pltpu.sync_copy(len_hbm.at[pl.ds(wid * 16, 16)], len_v)
    len_b = len_v[...][0]

    nd = jnp.clip(len_b - i0, 0, HALF_LEN)
    has_bnd = (nd % CHUNK) != 0
    kb = nd // CHUNK

    def stage_rows(dst, dst_off, first_row):
        for h in range(2):
            for j in range(HB // 16):
                rows = first_row + h * HB + j * 16 + lane
                idxs[h][pl.ds(j * 16, 16)] = rows + 1
            pltpu.async_copy(table_hbm.at[idxs[h]], bufs[h], gsem)
        for h in range(2):
            pltpu.make_async_copy(table_hbm.at[idxs[h]], bufs[h], gsem).wait()
            pltpu.sync_copy(bufs[h], dst.at[pl.ds(dst_off + h * HB, HB)])

    @pl.when(s < SPROWS // CHUNK)
    def _():
        stage_rows(sp_data, s * CHUNK, i0 + s * CHUNK)

    @pl.when(s == 15)
    def _():
        for j in range(HB // 16):
            idxs[0][pl.ds(j * 16, 16)] = jnp.zeros((16,), jnp.int32)
        pltpu.async_copy(table_hbm.at[idxs[0]], bufs[0], gsem).wait()
        pltpu.sync_copy(bufs[0], sp_pad.at[pl.ds(0, HB)])
        pltpu.sync_copy(bufs[0], sp_pad.at[pl.ds(HB, HB)])

    @pl.when(has_bnd)
    def _():
        for h in range(CHUNK // HB):
            for j in range(HB // 16):
                rows = i0 + kb * CHUNK + h * HB + j * 16 + lane
                idxs[h][pl.ds(j * 16, 16)] = jnp.where(rows < len_b, rows + 1,
                                                       0)
            pltpu.async_copy(table_hbm.at[idxs[h]], bufs[h], gsem)

    plsc.subcore_barrier()

    def chunk_ops(k, sp_base, fire):
        local_i = k * CHUNK
        row_out = row_out0 + local_i
        is_data = local_i + CHUNK <= nd
        is_pad = local_i >= nd

        @pl.when(is_data)
        def _():
            cp = pltpu.make_async_copy(
                sp_data.at[pl.ds(local_i - sp_base, CHUNK)],
                out_hbm.at[pl.ds(row_out, CHUNK)], csem)
            cp.start() if fire else cp.wait()

        @pl.when(jnp.logical_and(~is_data, is_pad))
        def _():
            cp = pltpu.make_async_copy(sp_pad,
                                       out_hbm.at[pl.ds(row_out, CHUNK)], csem)
            cp.start() if fire else cp.wait()

    def fire0(k, _):
        chunk_ops(k, 0, fire=True)
        return _

    def drain0(k, _):
        chunk_ops(k, 0, fire=False)
        return _

    p0 = PHASE_CHUNKS[0]
    lax.fori_loop(p0[0], p0[-1] + 1, fire0, None)
    lax.fori_loop(p0[0], p0[-1] + 1, drain0, None)

    @pl.when(has_bnd)
    def _():
        row_out = row_out0 + kb * CHUNK
        for h in range(CHUNK // HB):
            pltpu.make_async_copy(table_hbm.at[idxs[h]], bufs[h],
                                  gsem).wait()
            pltpu.async_copy(bufs[h], out_hbm.at[pl.ds(row_out + h * HB, HB)],
                             csem)
        for h in range(CHUNK // HB):
            pltpu.make_async_copy(bufs[h],
                                  out_hbm.at[pl.ds(row_out + h * HB, HB)],
                                  csem).wait()

    plsc.subcore_barrier()

    @pl.when(s < (HALF_LEN - SPROWS) // CHUNK)
    def _():
        stage_rows(sp_data, s * CHUNK, i0 + SPROWS + s * CHUNK)

    plsc.subcore_barrier()

    def fire1(k, _):
        chunk_ops(k, SPROWS, fire=True)
        return _

    def drain1(k, _):
        chunk_ops(k, SPROWS, fire=False)
        return _

    p1 = PHASE_CHUNKS[1]
    lax.fori_loop(p1[0], p1[-1] + 1, fire1, None)
    lax.fori_loop(p1[0], p1[-1] + 1, drain1, None)


def kernel(input_len, table):
    len_rep = jnp.repeat(input_len, 2 * 16)
    out = _pos_enc_sc(len_rep, table)
    return out.reshape(B, MAX_LEN, D)


# speedup vs baseline: 1.0084x; 1.0084x over previous
import functools

import jax
import jax.numpy as jnp
from jax import lax
from jax.experimental import pallas as pl
from jax.experimental.pallas import tpu as pltpu
from jax.experimental.pallas import tpu_sc as plsc

B = 16
MAX_LEN = 2048
D = 1024

NC = 2
NS = 16
HALF_LEN = MAX_LEN // NC
CHUNK = 64
NCHUNK = HALF_LEN // CHUNK
HB = 32
SPROWS = 896
PHASE_CHUNKS = (range(0, SPROWS // CHUNK), range(SPROWS // CHUNK, NCHUNK))

_mesh = plsc.VectorSubcoreMesh(core_axis_name="c", subcore_axis_name="s")


@functools.partial(
    pl.kernel,
    mesh=_mesh,
    out_type=jax.ShapeDtypeStruct((B * MAX_LEN, D), jnp.float32),
    scratch_types=[
        pltpu.VMEM((16,), jnp.int32),
        pltpu.VMEM((HB,), jnp.int32),
        pltpu.VMEM((HB,), jnp.int32),
        pltpu.VMEM((HB, D), jnp.float32),
        pltpu.VMEM((HB, D), jnp.float32),
        pltpu.VMEM_SHARED((SPROWS, D), jnp.float32),
        pltpu.VMEM_SHARED((CHUNK, D), jnp.float32),
        pltpu.SemaphoreType.DMA,
        pltpu.SemaphoreType.DMA,
        pltpu.SemaphoreType.DMA,
    ],
)
def _pos_enc_sc(len_hbm, table_hbm, tshift_hbm, pad_hbm, out_hbm, len_v,
                idx_a, idx_b, buf_a, buf_b, sp_data, sp_pad, stsem, gsem,
                csem):
    c = lax.axis_index("c")
    s = lax.axis_index("s")
    wid = s * NC + c
    i0 = c * HALF_LEN
    row_out0 = s * MAX_LEN + i0

    lane = lax.iota(jnp.int32, 16)

    pltpu.sync_copy(len_hbm.at[pl.ds(wid * 16, 16)], len_v)
    len_b = len_v[...][0]

    nd = jnp.clip(len_b - i0, 0, HALF_LEN)
    has_bnd = (nd % CHUNK) != 0
    kb = nd // CHUNK

    idxs = (idx_a, idx_b)
    bufs = (buf_a, buf_b)

    @pl.when(has_bnd)
    def _():
        for h in range(CHUNK // HB):
            for j in range(HB // 16):
                rows = i0 + kb * CHUNK + h * HB + j * 16 + lane
                idxs[h][pl.ds(j * 16, 16)] = jnp.where(rows < len_b, rows + 1,
                                                       0)
            pltpu.async_copy(table_hbm.at[idxs[h]], bufs[h], gsem)

    def chunk_ops(k, sp_base, fire):
        local_i = k * CHUNK
        row_out = row_out0 + local_i
        is_data = local_i + CHUNK <= nd
        is_pad = local_i >= nd

        @pl.when(is_data)
        def _():
            cp = pltpu.make_async_copy(
                sp_data.at[pl.ds(local_i - sp_base, CHUNK)],
                out_hbm.at[pl.ds(row_out, CHUNK)], csem)
            cp.start() if fire else cp.wait()

        @pl.when(jnp.logical_and(~is_data, is_pad))
        def _():
            cp = pltpu.make_async_copy(sp_pad,
                                       out_hbm.at[pl.ds(row_out, CHUNK)], csem)
            cp.start() if fire else cp.wait()

    for p, chunks in enumerate(PHASE_CHUNKS):
        sp_base = chunks[0] * CHUNK
        rows_p = len(chunks) * CHUNK

        @pl.when(s == 0)
        def _():
            pltpu.async_copy(tshift_hbm.at[pl.ds(i0 + sp_base, rows_p)],
                             sp_data.at[pl.ds(0, rows_p)], stsem)
            if p == 0:
                pltpu.async_copy(pad_hbm, sp_pad, stsem)
                pltpu.make_async_copy(pad_hbm, sp_pad, stsem).wait()
            pltpu.make_async_copy(tshift_hbm.at[pl.ds(i0 + sp_base, rows_p)],
                                  sp_data.at[pl.ds(0, rows_p)], stsem).wait()

        plsc.subcore_barrier()

        def fire_body(k, _, sp_base=sp_base):
            chunk_ops(k, sp_base, fire=True)
            return _

        def drain_body(k, _, sp_base=sp_base):
            chunk_ops(k, sp_base, fire=False)
            return _

        lax.fori_loop(chunks[0], chunks[-1] + 1, fire_body, None)
        lax.fori_loop(chunks[0], chunks[-1] + 1, drain_body, None)

        plsc.subcore_barrier()

    @pl.when(has_bnd)
    def _():
        row_out = row_out0 + kb * CHUNK
        for h in range(CHUNK // HB):
            pltpu.make_async_copy(table_hbm.at[idxs[h]], bufs[h],
                                  gsem).wait()
            pltpu.async_copy(bufs[h], out_hbm.at[pl.ds(row_out + h * HB, HB)],
                             csem)
        for h in range(CHUNK // HB):
            pltpu.make_async_copy(bufs[h],
                                  out_hbm.at[pl.ds(row_out + h * HB, HB)],
                                  csem).wait()


def kernel(input_len, table):
    tshift = lax.slice(table, (1, 0), (MAX_LEN + 1, D))
    pad_blk = jnp.broadcast_to(table[0], (CHUNK, D))
    len_rep = jnp.repeat(input_len, 2 * 16)
    out = _pos_enc_sc(len_rep, table, tshift, pad_blk)
    return out.reshape(B, MAX_LEN, D)
